# Initial kernel scaffold; baseline (speedup 1.0000x reference)
import jax
import jax.numpy as jnp
from jax.experimental import pallas as pl


def kernel(x, edge_index, W_mu, b_mu, W_logstd, b_logstd):
    raise NotImplementedError("write your pallas kernel here")


# speedup vs baseline: 17.1402x; 17.1402x over previous
import functools

import jax
import jax.numpy as jnp
from jax import lax
from jax.experimental import pallas as pl
from jax.experimental.pallas import tpu as pltpu
from jax.experimental.pallas import tpu_sc as plsc

N = 10000
E = 160000
C = 256
CH = C // 2
K = 128
NCHUNK = E // K
NSUB = 16
NCORE = 2
ROWS_BIG, ROWS_SMALL = 632, 624

_mesh = lambda: plsc.VectorSubcoreMesh(core_axis_name="c", subcore_axis_name="s")


def _row_base(s):
    return ROWS_SMALL * s + 8 * jnp.minimum(s, 2)


def _zero_fill(zbuf, ncols16):
    def row(i, _):
        def col(q, _):
            zbuf[i, pl.ds(q * 16, 16)] = jnp.zeros((16,), jnp.float32)
            return 0
        lax.fori_loop(0, ncols16, col, 0)
        return 0
    lax.fori_loop(0, 8, row, 0)


def _zero_rows(zbuf, acc, s):
    base = _row_base(s)
    ntrips = jnp.where(s < 2, ROWS_BIG // 8, ROWS_SMALL // 8)
    def zcopy(j, _):
        pltpu.sync_copy(zbuf, acc.at[pl.ds(base + j * 8, 8)])
        return 0
    lax.fori_loop(0, ntrips, zcopy, 0)


def _copy_out(acc, out_hbm, c, s):
    base = _row_base(s)
    @pl.when(s < 2)
    def _():
        pltpu.sync_copy(acc.at[pl.ds(base, ROWS_BIG)],
                        out_hbm.at[c, pl.ds(base, ROWS_BIG)])
    @pl.when(s >= 2)
    def _():
        pltpu.sync_copy(acc.at[pl.ds(base, ROWS_SMALL)],
                        out_hbm.at[c, pl.ds(base, ROWS_SMALL)])


@functools.partial(
    pl.kernel,
    out_type=jax.ShapeDtypeStruct((NCORE, 1, N), jnp.float32),
    mesh=_mesh(),
    scratch_types=[
        pltpu.VMEM((K,), jnp.int32),
        pltpu.VMEM((K,), jnp.float32),
        pltpu.VMEM((2000,), jnp.float32),
        pltpu.VMEM_SHARED((N,), jnp.float32),
    ],
)
def _deg_kernel(dst_hbm, out_hbm, dst_v, ones_v, zbuf, acc):
    c = lax.axis_index("c")
    s = lax.axis_index("s")
    w = s * NCORE + c

    def fill_ones(i, _):
        ones_v[pl.ds(i * 16, 16)] = jnp.ones((16,), jnp.float32)
        return 0
    lax.fori_loop(0, K // 16, fill_ones, 0)
    def fill_z(i, _):
        zbuf[pl.ds(i * 16, 16)] = jnp.zeros((16,), jnp.float32)
        return 0
    lax.fori_loop(0, 125, fill_z, 0)
    @pl.when(s == 0)
    def _():
        def zc(j, _):
            pltpu.sync_copy(zbuf, acc.at[pl.ds(j * 2000, 2000)])
            return 0
        lax.fori_loop(0, N // 2000, zc, 0)
    plsc.subcore_barrier()

    ntrips = (NCHUNK // (NSUB * NCORE)) + jnp.where(w < NCHUNK % (NSUB * NCORE), 1, 0)
    def body(j, _):
        cidx = w + (NSUB * NCORE) * j
        pltpu.sync_copy(dst_hbm.at[cidx, 0], dst_v)
        pltpu.sync_copy(ones_v, acc.at[dst_v], add=True)
        return 0
    lax.fori_loop(0, ntrips, body, 0)
    plsc.subcore_barrier()

    @pl.when(s == 0)
    def _():
        pltpu.sync_copy(acc, out_hbm.at[c, 0])


@functools.partial(
    pl.kernel,
    out_type=jax.ShapeDtypeStruct((NCORE, N, CH), jnp.float32),
    mesh=_mesh(),
    scratch_types=[
        pltpu.VMEM((K,), jnp.int32),
        pltpu.VMEM((K,), jnp.int32),
        pltpu.VMEM((K, CH), jnp.float32),
        pltpu.VMEM((8, CH), jnp.float32),
        pltpu.VMEM_SHARED((N, CH), jnp.float32),
        pltpu.SemaphoreType.DMA,
    ],
)
def _agg_kernel(xs_hbm, src_hbm, dst_hbm, out_hbm,
                src_v, dst_v, rows_v, zbuf, acc, sem):
    c = lax.axis_index("c")
    s = lax.axis_index("s")

    _zero_fill(zbuf, CH // 16)
    _zero_rows(zbuf, acc, s)
    plsc.subcore_barrier()

    off = c * N
    ntrips = (NCHUNK // NSUB) + jnp.where(s < NCHUNK % NSUB, 1, 0)
    def body(j, _):
        cidx = s + NSUB * j
        pltpu.sync_copy(src_hbm.at[cidx, 0], src_v)
        pltpu.sync_copy(dst_hbm.at[cidx, 0], dst_v)
        def addoff(i, _):
            src_v[pl.ds(i * 16, 16)] = src_v[pl.ds(i * 16, 16)] + off
            return 0
        lax.fori_loop(0, K // 16, addoff, 0)
        pltpu.async_copy(xs_hbm.at[src_v], rows_v, sem).wait()
        pltpu.sync_copy(rows_v, acc.at[dst_v], add=True)
        return 0
    lax.fori_loop(0, ntrips, body, 0)
    plsc.subcore_barrier()

    _copy_out(acc, out_hbm, c, s)


_TC_ROWS = 1000


def _dis_block(degp_ref):
    deg = degp_ref[0] + degp_ref[1] + 1.0
    return deg, lax.rsqrt(deg)


def _scale_body(degp_ref, x_ref, xs_ref):
    _, dis = _dis_block(degp_ref)
    xs = x_ref[...] * dis
    xs_ref[0] = xs[:, :CH]
    xs_ref[1] = xs[:, CH:]


def _scale(degp, x):
    grid = N // _TC_ROWS
    return pl.pallas_call(
        _scale_body,
        grid=(grid,),
        in_specs=[
            pl.BlockSpec((NCORE, _TC_ROWS, 1), lambda i: (0, i, 0)),
            pl.BlockSpec((_TC_ROWS, C), lambda i: (i, 0)),
        ],
        out_specs=pl.BlockSpec((NCORE, _TC_ROWS, CH), lambda i: (0, i, 0)),
        out_shape=jax.ShapeDtypeStruct((NCORE, N, CH), jnp.float32),
    )(degp, x)


def _final_body(degp_ref, x_ref, zp_ref, w_ref, b_ref, out_ref):
    deg, dis = _dis_block(degp_ref)
    recip = 1.0 / deg
    x = x_ref[...]
    zlo = dis * zp_ref[0] + recip * x[:, :CH]
    zhi = dis * zp_ref[1] + recip * x[:, CH:]
    out_ref[...] = (
        jnp.dot(zlo, w_ref[:CH, :], preferred_element_type=jnp.float32)
        + jnp.dot(zhi, w_ref[CH:, :], preferred_element_type=jnp.float32)
        + b_ref[...]
    )


def _final(degp, x, zp, wcat, bcat):
    grid = N // _TC_ROWS
    return pl.pallas_call(
        _final_body,
        grid=(grid,),
        in_specs=[
            pl.BlockSpec((NCORE, _TC_ROWS, 1), lambda i: (0, i, 0)),
            pl.BlockSpec((_TC_ROWS, C), lambda i: (i, 0)),
            pl.BlockSpec((NCORE, _TC_ROWS, CH), lambda i: (0, i, 0)),
            pl.BlockSpec((C, 2 * C), lambda i: (0, 0)),
            pl.BlockSpec((1, 2 * C), lambda i: (0, 0)),
        ],
        out_specs=pl.BlockSpec((_TC_ROWS, 2 * C), lambda i: (i, 0)),
        out_shape=jax.ShapeDtypeStruct((N, 2 * C), jnp.float32),
    )(degp, x, zp, wcat, bcat)


def kernel(x, edge_index, W_mu, b_mu, W_logstd, b_logstd):
    src = edge_index[0].astype(jnp.int32).reshape(NCHUNK, 1, K)
    dst = edge_index[1].astype(jnp.int32).reshape(NCHUNK, 1, K)

    degp = _deg_kernel(dst).reshape(NCORE, N, 1)
    xs2 = _scale(degp, x)
    zp = _agg_kernel(xs2.reshape(2 * N, CH), src, dst)

    wcat = jnp.concatenate([W_mu, W_logstd], axis=1)
    bcat = jnp.concatenate([b_mu, b_logstd]).reshape(1, 2 * C)
    out = _final(degp, x, zp, wcat, bcat)
    return out[:, :C], out[:, C:]


# speedup vs baseline: 34.3317x; 2.0030x over previous
import functools

import jax
import jax.numpy as jnp
from jax import lax
from jax.experimental import pallas as pl
from jax.experimental.pallas import tpu as pltpu
from jax.experimental.pallas import tpu_sc as plsc

N = 10000
E = 160000
C = 256
CH = C // 2
K = 128
NCHUNK = E // K
NSUB = 16
NCORE = 2
CPT = NCHUNK // NSUB - (NCHUNK % NSUB > 0)
CPT = 78
CPW = 39
NBUF = 2
ROWS_BIG, ROWS_SMALL = 632, 624

_mesh = lambda: plsc.VectorSubcoreMesh(core_axis_name="c", subcore_axis_name="s")


def _row_base(s):
    return ROWS_SMALL * s + 8 * jnp.minimum(s, 2)


@functools.partial(
    pl.kernel,
    out_type=jax.ShapeDtypeStruct((NCORE, 1, N), jnp.float32),
    mesh=_mesh(),
    scratch_types=[
        pltpu.VMEM((CPW + 1, 1, K), jnp.int32),
        pltpu.VMEM((CPW + 1, 1, K), jnp.int32),
        pltpu.VMEM((K,), jnp.float32),
        pltpu.VMEM((2000,), jnp.float32),
        pltpu.VMEM_SHARED((N,), jnp.float32),
        pltpu.SemaphoreType.DMA,
        pltpu.SemaphoreType.DMA,
    ],
)
def _deg_kernel(ed_hbm, out_hbm, ed_slab, dst_slab, ones_v, zbuf, acc,
                semd, semz):
    c = lax.axis_index("c")
    s = lax.axis_index("s")
    w = s * NCORE + c

    base = w * CPW
    pltpu.sync_copy(ed_hbm.at[pl.ds(base, CPW)], ed_slab.at[pl.ds(0, CPW)])
    @pl.when(w < 2)
    def _():
        pltpu.sync_copy(ed_hbm.at[pl.ds(NSUB * NCORE * CPW + w, 1)],
                        ed_slab.at[pl.ds(CPW, 1)])
    def unpack(i, _):
        sl = pl.ds((i % 8) * 16, 16)
        dst_slab[i // 8, 0, sl] = lax.shift_right_logical(ed_slab[i // 8, 0, sl], 14)
        return 0
    lax.fori_loop(0, (CPW + 1) * (K // 16), unpack, 0)

    def fill_ones(i, _):
        ones_v[pl.ds(i * 16, 16)] = jnp.ones((16,), jnp.float32)
        return 0
    lax.fori_loop(0, K // 16, fill_ones, 0)
    def fill_z(i, _):
        zbuf[pl.ds(i * 16, 16)] = jnp.zeros((16,), jnp.float32)
        return 0
    lax.fori_loop(0, 125, fill_z, 0)
    @pl.when(s == 0)
    def _():
        def zfire(j, _):
            pltpu.async_copy(zbuf, acc.at[pl.ds(j * 2000, 2000)], semz)
            return 0
        lax.fori_loop(0, N // 2000, zfire, 0)
        def zdrain(j, _):
            pltpu.make_async_copy(zbuf, acc.at[pl.ds(0, 2000)], semz).wait()
            return 0
        lax.fori_loop(0, N // 2000, zdrain, 0)
    plsc.subcore_barrier()

    nch = CPW + jnp.where(w < 2, 1, 0)
    def fire(j, _):
        pltpu.async_copy(ones_v, acc.at[dst_slab.at[j, 0]], semd, add=True)
        return 0
    lax.fori_loop(0, nch, fire, 0)
    def drain(j, _):
        pltpu.make_async_copy(ones_v, acc.at[dst_slab.at[0, 0]], semd).wait()
        return 0
    lax.fori_loop(0, nch, drain, 0)
    plsc.subcore_barrier()

    @pl.when(s == 0)
    def _():
        pltpu.sync_copy(acc, out_hbm.at[c, 0])


@functools.partial(
    pl.kernel,
    out_type=jax.ShapeDtypeStruct((NCORE, N, CH), jnp.float32),
    mesh=_mesh(),
    scratch_types=[
        pltpu.VMEM((CPT + 1, 1, K), jnp.int32),
        pltpu.VMEM((NBUF, 1, K), jnp.int32),
        pltpu.VMEM((NBUF, 1, K), jnp.int32),
        pltpu.VMEM((K, CH), jnp.float32),
        pltpu.VMEM((K, CH), jnp.float32),
        pltpu.VMEM((8, CH), jnp.float32),
        pltpu.VMEM_SHARED((N, CH), jnp.float32),
        pltpu.SemaphoreType.DMA,
        pltpu.SemaphoreType.DMA,
        pltpu.SemaphoreType.DMA,
        pltpu.SemaphoreType.DMA,
        pltpu.SemaphoreType.DMA,
    ],
)
def _agg_kernel(xs_hbm, ed_hbm, out_hbm,
                ed_slab, src_v, dst_v, r0, r1, zbuf, acc,
                g0, g1, s0, s1, semz):
    rows = [r0, r1]
    semg = [g0, g1]
    sems = [s0, s1]
    c = lax.axis_index("c")
    s = lax.axis_index("s")

    base = s * CPT
    pltpu.sync_copy(ed_hbm.at[pl.ds(base, CPT)], ed_slab.at[pl.ds(0, CPT)])
    @pl.when(s < 2)
    def _():
        pltpu.sync_copy(ed_hbm.at[pl.ds(NSUB * CPT + s, 1)],
                        ed_slab.at[pl.ds(CPT, 1)])

    off = c * N
    def unpack(j, b):
        def go(i, _):
            sl = pl.ds(i * 16, 16)
            ed = ed_slab[j, 0, sl]
            src_v[b, 0, sl] = (ed & 0x3FFF) + off
            dst_v[b, 0, sl] = lax.shift_right_logical(ed, 14)
            return 0
        lax.fori_loop(0, K // 16, go, 0)

    def fill_z(i, _):
        zbuf[i // 8, pl.ds((i % 8) * 16, 16)] = jnp.zeros((16,), jnp.float32)
        return 0
    lax.fori_loop(0, 8 * (CH // 16), fill_z, 0)
    rbase = _row_base(s)
    nz = jnp.where(s < 2, ROWS_BIG // 8, ROWS_SMALL // 8)
    def zfire(j, _):
        pltpu.async_copy(zbuf, acc.at[pl.ds(rbase + j * 8, 8)], semz)
        return 0
    lax.fori_loop(0, nz, zfire, 0)
    def zdrain(j, _):
        pltpu.make_async_copy(zbuf, acc.at[pl.ds(rbase, 8)], semz).wait()
        return 0
    lax.fori_loop(0, nz, zdrain, 0)
    plsc.subcore_barrier()

    def gfire(j, b):
        pltpu.async_copy(xs_hbm.at[src_v.at[b, 0]], rows[b], semg[b])
    def gwait(b):
        pltpu.make_async_copy(xs_hbm.at[src_v.at[b, 0]], rows[b], semg[b]).wait()
    def sfire(b):
        pltpu.async_copy(rows[b], acc.at[dst_v.at[b, 0]], sems[b], add=True)
    def swait(b):
        pltpu.make_async_copy(rows[b], acc.at[dst_v.at[b, 0]], sems[b]).wait()

    unpack(0, 0)
    gfire(0, 0)
    unpack(1, 1)
    gfire(1, 1)
    gwait(0)
    sfire(0)
    def steady(g, _):
        for b in range(NBUF):
            j = g * NBUF + b
            swait(b)
            unpack(j, b)
            gfire(j, b)
            gwait(1 - b)
            sfire(1 - b)
        return 0
    lax.fori_loop(1, CPT // NBUF, steady, 0)
    gwait(1)
    sfire(1)
    swait(0)
    swait(1)
    @pl.when(s < 2)
    def _():
        unpack(CPT, 0)
        pltpu.async_copy(xs_hbm.at[src_v.at[0, 0]], rows[0], semg[0]).wait()
        pltpu.sync_copy(rows[0], acc.at[dst_v.at[0, 0]], add=True)
    plsc.subcore_barrier()

    rbig = pl.ds(rbase, ROWS_BIG)
    rsml = pl.ds(rbase, ROWS_SMALL)
    @pl.when(s < 2)
    def _():
        pltpu.sync_copy(acc.at[rbig], out_hbm.at[c, rbig])
    @pl.when(s >= 2)
    def _():
        pltpu.sync_copy(acc.at[rsml], out_hbm.at[c, rsml])


_TC_ROWS = 1000


def _dis_block(degp_ref):
    deg = degp_ref[0] + degp_ref[1] + 1.0
    return deg, lax.rsqrt(deg)


def _scale_body(degp_ref, x_ref, xs_ref):
    _, dis = _dis_block(degp_ref)
    xs = x_ref[...] * dis
    xs_ref[0] = xs[:, :CH]
    xs_ref[1] = xs[:, CH:]


def _scale(degp, x):
    grid = N // _TC_ROWS
    return pl.pallas_call(
        _scale_body,
        grid=(grid,),
        in_specs=[
            pl.BlockSpec((NCORE, _TC_ROWS, 1), lambda i: (0, i, 0)),
            pl.BlockSpec((_TC_ROWS, C), lambda i: (i, 0)),
        ],
        out_specs=pl.BlockSpec((NCORE, _TC_ROWS, CH), lambda i: (0, i, 0)),
        out_shape=jax.ShapeDtypeStruct((NCORE, N, CH), jnp.float32),
    )(degp, x)


def _final_body(degp_ref, x_ref, zp_ref, wmu_ref, wls_ref, bmu_ref, bls_ref,
                mu_ref, ls_ref):
    deg, dis = _dis_block(degp_ref)
    recip = 1.0 / deg
    x = x_ref[...]
    zlo = dis * zp_ref[0] + recip * x[:, :CH]
    zhi = dis * zp_ref[1] + recip * x[:, CH:]
    mu_ref[...] = (
        jnp.dot(zlo, wmu_ref[:CH, :], preferred_element_type=jnp.float32)
        + jnp.dot(zhi, wmu_ref[CH:, :], preferred_element_type=jnp.float32)
        + bmu_ref[...]
    )
    ls_ref[...] = (
        jnp.dot(zlo, wls_ref[:CH, :], preferred_element_type=jnp.float32)
        + jnp.dot(zhi, wls_ref[CH:, :], preferred_element_type=jnp.float32)
        + bls_ref[...]
    )


def _final(degp, x, zp, wmu, wls, bmu, bls):
    grid = N // _TC_ROWS
    return pl.pallas_call(
        _final_body,
        grid=(grid,),
        in_specs=[
            pl.BlockSpec((NCORE, _TC_ROWS, 1), lambda i: (0, i, 0)),
            pl.BlockSpec((_TC_ROWS, C), lambda i: (i, 0)),
            pl.BlockSpec((NCORE, _TC_ROWS, CH), lambda i: (0, i, 0)),
            pl.BlockSpec((C, C), lambda i: (0, 0)),
            pl.BlockSpec((C, C), lambda i: (0, 0)),
            pl.BlockSpec((1, C), lambda i: (0, 0)),
            pl.BlockSpec((1, C), lambda i: (0, 0)),
        ],
        out_specs=[
            pl.BlockSpec((_TC_ROWS, C), lambda i: (i, 0)),
            pl.BlockSpec((_TC_ROWS, C), lambda i: (i, 0)),
        ],
        out_shape=[
            jax.ShapeDtypeStruct((N, C), jnp.float32),
            jax.ShapeDtypeStruct((N, C), jnp.float32),
        ],
    )(degp, x, zp, wmu, wls, bmu, bls)


def kernel(x, edge_index, W_mu, b_mu, W_logstd, b_logstd):
    src = edge_index[0].astype(jnp.int32)
    dst = edge_index[1].astype(jnp.int32)
    ed = (src | (dst << 14)).reshape(NCHUNK, 1, K)

    degp = _deg_kernel(ed).reshape(NCORE, N, 1)
    xs2 = _scale(degp, x)
    zp = _agg_kernel(xs2.reshape(2 * N, CH), ed)
    mu, ls = _final(degp, x, zp, W_mu, W_logstd,
                    b_mu.reshape(1, C), b_logstd.reshape(1, C))
    return mu, ls
